# Initial kernel scaffold; baseline (speedup 1.0000x reference)
import jax
import jax.numpy as jnp
from jax.experimental import pallas as pl


def kernel(x, edge_index, Wl1, bl1, Wr1, Wl2, bl2, Wr2, Wlin, blin):
    raise NotImplementedError("write your pallas kernel here")


# speedup vs baseline: 1.0020x; 1.0020x over previous
import jax
import jax.numpy as jnp
from jax.experimental import pallas as pl
from jax.experimental.pallas import tpu as pltpu

N_NODES = 10000
BN = 1000


def _layer1_body(cnt_ref, agg_ref, x_ref, wlt_ref, wrt_ref, bl_ref, o_ref):
    inv = 1.0 / jnp.clip(cnt_ref[...], 1.0, None)
    mean = agg_ref[...] * inv
    h = jnp.dot(mean, wlt_ref[...], preferred_element_type=jnp.float32)
    h = h + jnp.dot(x_ref[...], wrt_ref[...], preferred_element_type=jnp.float32)
    h = h + bl_ref[...]
    o_ref[...] = jnp.maximum(h, 0.0)


def _layer2_body(cnt_ref, agg_ref, h_ref, wlt_ref, wrt_ref, bl_ref,
                 wlint_ref, blin_ref, o_ref):
    inv = 1.0 / jnp.clip(cnt_ref[...], 1.0, None)
    mean = agg_ref[...] * inv
    h2 = jnp.dot(mean, wlt_ref[...], preferred_element_type=jnp.float32)
    h2 = h2 + jnp.dot(h_ref[...], wrt_ref[...], preferred_element_type=jnp.float32)
    h2 = jnp.maximum(h2 + bl_ref[...], 0.0)
    out = jnp.dot(h2, wlint_ref[...], preferred_element_type=jnp.float32)
    o_ref[...] = out + blin_ref[...]


def _dense_layer1(cnt, agg, x, WlT, WrT, bl):
    din, h = WlT.shape
    return pl.pallas_call(
        _layer1_body,
        grid=(N_NODES // BN,),
        in_specs=[
            pl.BlockSpec((BN, 1), lambda i: (i, 0)),
            pl.BlockSpec((BN, din), lambda i: (i, 0)),
            pl.BlockSpec((BN, din), lambda i: (i, 0)),
            pl.BlockSpec((din, h), lambda i: (0, 0)),
            pl.BlockSpec((din, h), lambda i: (0, 0)),
            pl.BlockSpec((1, h), lambda i: (0, 0)),
        ],
        out_specs=pl.BlockSpec((BN, h), lambda i: (i, 0)),
        out_shape=jax.ShapeDtypeStruct((N_NODES, h), jnp.float32),
    )(cnt, agg, x, WlT, WrT, bl)


def _dense_layer2(cnt, agg, h1, Wl2T, Wr2T, bl2, WlinT, blin):
    din, h = Wl2T.shape
    out = WlinT.shape[1]
    return pl.pallas_call(
        _layer2_body,
        grid=(N_NODES // BN,),
        in_specs=[
            pl.BlockSpec((BN, 1), lambda i: (i, 0)),
            pl.BlockSpec((BN, din), lambda i: (i, 0)),
            pl.BlockSpec((BN, din), lambda i: (i, 0)),
            pl.BlockSpec((din, h), lambda i: (0, 0)),
            pl.BlockSpec((din, h), lambda i: (0, 0)),
            pl.BlockSpec((1, h), lambda i: (0, 0)),
            pl.BlockSpec((h, out), lambda i: (0, 0)),
            pl.BlockSpec((1, out), lambda i: (0, 0)),
        ],
        out_specs=pl.BlockSpec((BN, out), lambda i: (i, 0)),
        out_shape=jax.ShapeDtypeStruct((N_NODES, out), jnp.float32),
    )(cnt, agg, h1, Wl2T, Wr2T, bl2, WlinT, blin)


def _segsum(table, src, dst):
    msgs = jnp.take(table, src, axis=0)
    return jax.ops.segment_sum(msgs, dst, num_segments=N_NODES)


def kernel(x, edge_index, Wl1, bl1, Wr1, Wl2, bl2, Wr2, Wlin, blin):
    src = edge_index[0].astype(jnp.int32)
    dst = edge_index[1].astype(jnp.int32)
    cnt = jax.ops.segment_sum(jnp.ones((src.shape[0],), jnp.float32), dst,
                              num_segments=N_NODES)[:, None]
    agg1 = _segsum(x, src, dst)
    h1 = _dense_layer1(cnt, agg1, x, Wl1.T, Wr1.T, bl1[None, :])
    agg2 = _segsum(h1, src, dst)
    return _dense_layer2(cnt, agg2, h1, Wl2.T, Wr2.T, bl2[None, :],
                         Wlin.T, blin[None, :])


# speedup vs baseline: 3.0816x; 3.0754x over previous
import functools

import jax
import jax.numpy as jnp
from jax import lax
from jax.experimental import pallas as pl
from jax.experimental.pallas import tpu as pltpu
from jax.experimental.pallas import tpu_sc as plsc

N_NODES = 10000
N_EDGES_TOTAL = 160000
NPAD = 10240
DB = 128
NS = 16
NC = 2
RPT = NPAD // NS
EPT = N_EDGES_TOTAL // NS
CH = 40
NJ = 25
NG = EPT // (NJ * CH)
BN = 1000


@functools.lru_cache(maxsize=None)
def _make_segsum(nb, with_counts):
    nbpc = nb // NC
    mesh = plsc.VectorSubcoreMesh(core_axis_name="c", subcore_axis_name="s")

    out_type = [jax.ShapeDtypeStruct((nb * NPAD, DB), jnp.float32)]
    if with_counts:
        out_type.append(jax.ShapeDtypeStruct((NPAD,), jnp.float32))

    scratch_types = [
        pltpu.VMEM((NJ, CH), jnp.int32),
        pltpu.VMEM((NJ, CH), jnp.int32),
        pltpu.VMEM((CH, DB), jnp.float32),
        pltpu.VMEM((48,), jnp.float32),
        pltpu.VMEM_SHARED((NPAD, DB), jnp.float32),
        pltpu.VMEM_SHARED((NPAD,), jnp.float32),
    ]

    @functools.partial(pl.kernel, mesh=mesh, out_type=tuple(out_type),
                       scratch_types=scratch_types)
    def segsum(tab, srch, dsth, *refs):
        if with_counts:
            out, cnt_out = refs[0], refs[1]
            scratch = refs[2:]
        else:
            out = refs[0]
            scratch = refs[1:]
        src_sc, dst_sc, rows, ones_v, acc, cnt_acc = scratch

        c = lax.axis_index("c")
        s = lax.axis_index("s")
        row0 = s * RPT

        for kk in range(3):
            ones_v[pl.ds(kk * 16, 16)] = jnp.ones((16,), jnp.float32)

        def zero_acc():
            def zrow(r, carry):
                for kk in range(DB // 16):
                    rows[r, pl.ds(kk * 16, 16)] = jnp.zeros((16,), jnp.float32)
                return carry
            lax.fori_loop(0, CH, zrow, 0)
            for j in range(RPT // CH):
                pltpu.sync_copy(rows, acc.at[pl.ds(row0 + j * CH, CH)])

        def zero_cnt():
            for j in range(RPT // DB):
                pltpu.sync_copy(rows.at[0], cnt_acc.at[pl.ds(row0 + j * DB, DB)])

        zero_acc()
        if with_counts:
            @pl.when(c == 0)
            def _():
                zero_cnt()

        for bb in range(nbpc):
            if bb > 0:
                zero_acc()
            plsc.subcore_barrier()

            b = c * nbpc + bb

            def gbody(g, carry):
                pltpu.sync_copy(srch.at[b, s, g], src_sc)
                pltpu.sync_copy(dsth.at[s, g], dst_sc)

                def body(j, carry2):
                    pltpu.sync_copy(tab.at[src_sc.at[j]], rows)
                    pltpu.sync_copy(rows, acc.at[dst_sc.at[j]], add=True)
                    return carry2
                lax.fori_loop(0, NJ, body, 0)
                return carry
            lax.fori_loop(0, NG, gbody, 0)

            if with_counts and bb == 0:
                @pl.when(c == 0)
                def _():
                    def cg(g, carry):
                        pltpu.sync_copy(dsth.at[s, g], dst_sc)

                        def cb(j, carry2):
                            pltpu.sync_copy(ones_v.at[pl.ds(0, CH)],
                                            cnt_acc.at[dst_sc.at[j]], add=True)
                            return carry2
                        lax.fori_loop(0, NJ, cb, 0)
                        return carry
                    lax.fori_loop(0, NG, cg, 0)

            plsc.subcore_barrier()

            brow = b * NPAD + row0
            for j in range(RPT // DB):
                pltpu.sync_copy(acc.at[pl.ds(row0 + j * DB, DB)],
                                out.at[pl.ds(brow + j * DB, DB)])
            if with_counts and bb == 0:
                @pl.when(c == 0)
                def _():
                    pltpu.sync_copy(cnt_acc.at[pl.ds(row0, RPT)],
                                    cnt_out.at[pl.ds(row0, RPT)])
            if bb + 1 < nbpc:
                plsc.subcore_barrier()

    return segsum


def _aggregate(table, srcb, dst4, nb, with_counts):
    res = _make_segsum(nb, with_counts)(table, srcb, dst4)
    if with_counts:
        aggflat, cnt = res
    else:
        aggflat = res[0] if isinstance(res, (tuple, list)) else res
    agg = aggflat.reshape(nb, NPAD, DB)[:, :N_NODES]
    agg = agg.transpose(1, 0, 2).reshape(N_NODES, nb * DB)
    if with_counts:
        return agg, cnt[:N_NODES, None]
    return agg


def _layer1_body(cnt_ref, agg_ref, x_ref, wlt_ref, wrt_ref, bl_ref, o_ref):
    inv = 1.0 / jnp.clip(cnt_ref[...], 1.0, None)
    mean = agg_ref[...] * inv
    h = jnp.dot(mean, wlt_ref[...], preferred_element_type=jnp.float32)
    h = h + jnp.dot(x_ref[...], wrt_ref[...], preferred_element_type=jnp.float32)
    h = h + bl_ref[...]
    o_ref[...] = jnp.maximum(h, 0.0)


def _layer2_body(cnt_ref, agg_ref, h_ref, wlt_ref, wrt_ref, bl_ref,
                 wlint_ref, blin_ref, o_ref):
    inv = 1.0 / jnp.clip(cnt_ref[...], 1.0, None)
    mean = agg_ref[...] * inv
    h2 = jnp.dot(mean, wlt_ref[...], preferred_element_type=jnp.float32)
    h2 = h2 + jnp.dot(h_ref[...], wrt_ref[...], preferred_element_type=jnp.float32)
    h2 = jnp.maximum(h2 + bl_ref[...], 0.0)
    out = jnp.dot(h2, wlint_ref[...], preferred_element_type=jnp.float32)
    o_ref[...] = out + blin_ref[...]


def _dense_layer1(cnt, agg, x, WlT, WrT, bl):
    din, h = WlT.shape
    return pl.pallas_call(
        _layer1_body,
        grid=(N_NODES // BN,),
        in_specs=[
            pl.BlockSpec((BN, 1), lambda i: (i, 0)),
            pl.BlockSpec((BN, din), lambda i: (i, 0)),
            pl.BlockSpec((BN, din), lambda i: (i, 0)),
            pl.BlockSpec((din, h), lambda i: (0, 0)),
            pl.BlockSpec((din, h), lambda i: (0, 0)),
            pl.BlockSpec((1, h), lambda i: (0, 0)),
        ],
        out_specs=pl.BlockSpec((BN, h), lambda i: (i, 0)),
        out_shape=jax.ShapeDtypeStruct((N_NODES, h), jnp.float32),
    )(cnt, agg, x, WlT, WrT, bl)


def _dense_layer2(cnt, agg, h1, Wl2T, Wr2T, bl2, WlinT, blin):
    din, h = Wl2T.shape
    out = WlinT.shape[1]
    return pl.pallas_call(
        _layer2_body,
        grid=(N_NODES // BN,),
        in_specs=[
            pl.BlockSpec((BN, 1), lambda i: (i, 0)),
            pl.BlockSpec((BN, din), lambda i: (i, 0)),
            pl.BlockSpec((BN, din), lambda i: (i, 0)),
            pl.BlockSpec((din, h), lambda i: (0, 0)),
            pl.BlockSpec((din, h), lambda i: (0, 0)),
            pl.BlockSpec((1, h), lambda i: (0, 0)),
            pl.BlockSpec((h, out), lambda i: (0, 0)),
            pl.BlockSpec((1, out), lambda i: (0, 0)),
        ],
        out_specs=pl.BlockSpec((BN, out), lambda i: (i, 0)),
        out_shape=jax.ShapeDtypeStruct((N_NODES, out), jnp.float32),
    )(cnt, agg, h1, Wl2T, Wr2T, bl2, WlinT, blin)


def kernel(x, edge_index, Wl1, bl1, Wr1, Wl2, bl2, Wr2, Wlin, blin):
    src = edge_index[0].astype(jnp.int32)
    dst = edge_index[1].astype(jnp.int32)
    dst4 = dst.reshape(NS, NG, NJ, CH)

    def src_blocks(nb):
        offs = (jnp.arange(nb, dtype=jnp.int32) * N_NODES)[:, None]
        return (src[None, :] + offs).reshape(nb, NS, NG, NJ, CH)

    nb1 = x.shape[1] // DB
    xt = x.reshape(N_NODES, nb1, DB).transpose(1, 0, 2).reshape(nb1 * N_NODES, DB)
    agg1, cnt = _aggregate(xt, src_blocks(nb1), dst4, nb1, True)
    h1 = _dense_layer1(cnt, agg1, x, Wl1.T, Wr1.T, bl1[None, :])

    nb2 = h1.shape[1] // DB
    ht = h1.reshape(N_NODES, nb2, DB).transpose(1, 0, 2).reshape(nb2 * N_NODES, DB)
    agg2 = _aggregate(ht, src_blocks(nb2), dst4, nb2, False)
    return _dense_layer2(cnt, agg2, h1, Wl2.T, Wr2.T, bl2[None, :],
                         Wlin.T, blin[None, :])


# speedup vs baseline: 4.4586x; 1.4469x over previous
import functools

import jax
import jax.numpy as jnp
from jax import lax
from jax.experimental import pallas as pl
from jax.experimental.pallas import tpu as pltpu
from jax.experimental.pallas import tpu_sc as plsc

N_NODES = 10000
N_EDGES_TOTAL = 160000
NPAD = 10240
DB = 128
NS = 16
NC = 2
RPT = NPAD // NS
EPT = N_EDGES_TOTAL // NS
CH = 40
NJ = 10
NG = EPT // (NJ * CH)
BN = 1000


@functools.lru_cache(maxsize=None)
def _make_segsum(nb, with_counts):
    nbpc = nb // NC
    mesh = plsc.VectorSubcoreMesh(core_axis_name="c", subcore_axis_name="s")

    out_type = [jax.ShapeDtypeStruct((nb * NPAD, DB), jnp.float32)]
    if with_counts:
        out_type.append(jax.ShapeDtypeStruct((NPAD,), jnp.float32))

    scratch_types = [
        pltpu.VMEM((2, NJ, CH), jnp.int32),
        pltpu.VMEM((CH, DB), jnp.float32),
        pltpu.VMEM((CH, DB), jnp.float32),
        pltpu.VMEM((48,), jnp.float32),
        pltpu.VMEM_SHARED((NPAD, DB), jnp.float32),
        pltpu.VMEM_SHARED((NPAD,), jnp.float32),
        pltpu.SemaphoreType.DMA,
        pltpu.SemaphoreType.DMA,
        pltpu.SemaphoreType.DMA,
        pltpu.SemaphoreType.DMA,
        pltpu.SemaphoreType.DMA,
    ]

    @functools.partial(pl.kernel, mesh=mesh, out_type=tuple(out_type),
                       scratch_types=scratch_types)
    def segsum(tab, idxh, *refs):
        if with_counts:
            out, cnt_out = refs[0], refs[1]
            scratch = refs[2:]
        else:
            out = refs[0]
            scratch = refs[1:]
        (idxbuf, rows0, rows1, ones_v, acc, cnt_acc,
         gsem0, gsem1, ssem0, ssem1, csem) = scratch
        bufs = (rows0, rows1)
        gsems = (gsem0, gsem1)
        ssems = (ssem0, ssem1)

        c = lax.axis_index("c")
        s = lax.axis_index("s")
        row0 = s * RPT

        for kk in range(3):
            ones_v[pl.ds(kk * 16, 16)] = jnp.ones((16,), jnp.float32)

        def zero_acc():
            def zrow(r, carry):
                for kk in range(DB // 16):
                    rows0[r, pl.ds(kk * 16, 16)] = jnp.zeros((16,), jnp.float32)
                return carry
            lax.fori_loop(0, CH, zrow, 0)
            for j in range(RPT // CH):
                pltpu.sync_copy(rows0, acc.at[pl.ds(row0 + j * CH, CH)])

        def zero_cnt():
            for j in range(RPT // DB):
                pltpu.sync_copy(rows0.at[0], cnt_acc.at[pl.ds(row0 + j * DB, DB)])

        zero_acc()
        if with_counts:
            @pl.when(c == 0)
            def _():
                zero_cnt()

        for bb in range(nbpc):
            if bb > 0:
                zero_acc()
            plsc.subcore_barrier()

            b = c * nbpc + bb
            do_counts = with_counts and bb == 0

            def gbody(g, carry):
                pltpu.sync_copy(idxh.at[b, s, g], idxbuf)
                gh = [None, None]
                sh = [None, None]
                chs = []
                gh[0] = pltpu.async_copy(tab.at[idxbuf.at[0, 0]], rows0, gsem0)
                gh[1] = pltpu.async_copy(tab.at[idxbuf.at[0, 1]], rows1, gsem1)
                for j in range(NJ):
                    bi = j % 2
                    gh[bi].wait()
                    sh[bi] = pltpu.async_copy(bufs[bi],
                                              acc.at[idxbuf.at[1, j]],
                                              ssems[bi], add=True)
                    if do_counts:
                        @pl.when(c == 0)
                        def _():
                            chs.append(pltpu.async_copy(
                                ones_v.at[pl.ds(0, CH)],
                                cnt_acc.at[idxbuf.at[1, j]], csem, add=True))
                    if j + 2 < NJ:
                        sh[bi].wait()
                        gh[bi] = pltpu.async_copy(tab.at[idxbuf.at[0, j + 2]],
                                                  bufs[bi], gsems[bi])
                sh[(NJ - 2) % 2].wait()
                sh[(NJ - 1) % 2].wait()
                if do_counts:
                    @pl.when(c == 0)
                    def _():
                        for ch in chs:
                            ch.wait()
                return carry
            lax.fori_loop(0, NG, gbody, 0)

            plsc.subcore_barrier()

            brow = b * NPAD + row0
            for j in range(RPT // DB):
                pltpu.sync_copy(acc.at[pl.ds(row0 + j * DB, DB)],
                                out.at[pl.ds(brow + j * DB, DB)])
            if with_counts and bb == 0:
                @pl.when(c == 0)
                def _():
                    pltpu.sync_copy(cnt_acc.at[pl.ds(row0, RPT)],
                                    cnt_out.at[pl.ds(row0, RPT)])
            if bb + 1 < nbpc:
                plsc.subcore_barrier()

    return segsum


def _aggregate(table, idxb, nb, with_counts):
    res = _make_segsum(nb, with_counts)(table, idxb)
    if with_counts:
        aggflat, cnt = res
    else:
        aggflat = res[0] if isinstance(res, (tuple, list)) else res
    agg = aggflat.reshape(nb, NPAD, DB)[:, :N_NODES]
    agg = agg.transpose(1, 0, 2).reshape(N_NODES, nb * DB)
    if with_counts:
        return agg, cnt[:N_NODES, None]
    return agg


def _layer1_body(cnt_ref, agg_ref, x_ref, wlt_ref, wrt_ref, bl_ref, o_ref):
    inv = 1.0 / jnp.clip(cnt_ref[...], 1.0, None)
    mean = agg_ref[...] * inv
    h = jnp.dot(mean, wlt_ref[...], preferred_element_type=jnp.float32)
    h = h + jnp.dot(x_ref[...], wrt_ref[...], preferred_element_type=jnp.float32)
    h = h + bl_ref[...]
    o_ref[...] = jnp.maximum(h, 0.0)


def _layer2_body(cnt_ref, agg_ref, h_ref, wlt_ref, wrt_ref, bl_ref,
                 wlint_ref, blin_ref, o_ref):
    inv = 1.0 / jnp.clip(cnt_ref[...], 1.0, None)
    mean = agg_ref[...] * inv
    h2 = jnp.dot(mean, wlt_ref[...], preferred_element_type=jnp.float32)
    h2 = h2 + jnp.dot(h_ref[...], wrt_ref[...], preferred_element_type=jnp.float32)
    h2 = jnp.maximum(h2 + bl_ref[...], 0.0)
    out = jnp.dot(h2, wlint_ref[...], preferred_element_type=jnp.float32)
    o_ref[...] = out + blin_ref[...]


def _dense_layer1(cnt, agg, x, WlT, WrT, bl):
    din, h = WlT.shape
    return pl.pallas_call(
        _layer1_body,
        grid=(N_NODES // BN,),
        in_specs=[
            pl.BlockSpec((BN, 1), lambda i: (i, 0)),
            pl.BlockSpec((BN, din), lambda i: (i, 0)),
            pl.BlockSpec((BN, din), lambda i: (i, 0)),
            pl.BlockSpec((din, h), lambda i: (0, 0)),
            pl.BlockSpec((din, h), lambda i: (0, 0)),
            pl.BlockSpec((1, h), lambda i: (0, 0)),
        ],
        out_specs=pl.BlockSpec((BN, h), lambda i: (i, 0)),
        out_shape=jax.ShapeDtypeStruct((N_NODES, h), jnp.float32),
    )(cnt, agg, x, WlT, WrT, bl)


def _dense_layer2(cnt, agg, h1, Wl2T, Wr2T, bl2, WlinT, blin):
    din, h = Wl2T.shape
    out = WlinT.shape[1]
    return pl.pallas_call(
        _layer2_body,
        grid=(N_NODES // BN,),
        in_specs=[
            pl.BlockSpec((BN, 1), lambda i: (i, 0)),
            pl.BlockSpec((BN, din), lambda i: (i, 0)),
            pl.BlockSpec((BN, din), lambda i: (i, 0)),
            pl.BlockSpec((din, h), lambda i: (0, 0)),
            pl.BlockSpec((din, h), lambda i: (0, 0)),
            pl.BlockSpec((1, h), lambda i: (0, 0)),
            pl.BlockSpec((h, out), lambda i: (0, 0)),
            pl.BlockSpec((1, out), lambda i: (0, 0)),
        ],
        out_specs=pl.BlockSpec((BN, out), lambda i: (i, 0)),
        out_shape=jax.ShapeDtypeStruct((N_NODES, out), jnp.float32),
    )(cnt, agg, h1, Wl2T, Wr2T, bl2, WlinT, blin)


def kernel(x, edge_index, Wl1, bl1, Wr1, Wl2, bl2, Wr2, Wlin, blin):
    src = edge_index[0].astype(jnp.int32)
    dst = edge_index[1].astype(jnp.int32)

    def idx_blocks(nb):
        offs = (jnp.arange(nb, dtype=jnp.int32) * N_NODES)[:, None]
        srcb = src[None, :] + offs
        dstb = jnp.broadcast_to(dst[None, :], (nb, N_EDGES_TOTAL))
        comb = jnp.stack([srcb, dstb], axis=1)
        comb = comb.reshape(nb, 2, NS, NG, NJ, CH)
        return comb.transpose(0, 2, 3, 1, 4, 5)

    nb1 = x.shape[1] // DB
    xt = x.reshape(N_NODES, nb1, DB).transpose(1, 0, 2).reshape(nb1 * N_NODES, DB)
    agg1, cnt = _aggregate(xt, idx_blocks(nb1), nb1, True)
    h1 = _dense_layer1(cnt, agg1, x, Wl1.T, Wr1.T, bl1[None, :])

    nb2 = h1.shape[1] // DB
    ht = h1.reshape(N_NODES, nb2, DB).transpose(1, 0, 2).reshape(nb2 * N_NODES, DB)
    agg2 = _aggregate(ht, idx_blocks(nb2), nb2, False)
    return _dense_layer2(cnt, agg2, h1, Wl2.T, Wr2.T, bl2[None, :],
                         Wlin.T, blin[None, :])


# speedup vs baseline: 4.8866x; 1.0960x over previous
import functools

import jax
import jax.numpy as jnp
from jax import lax
from jax.experimental import pallas as pl
from jax.experimental.pallas import tpu as pltpu
from jax.experimental.pallas import tpu_sc as plsc

N_NODES = 10000
N_EDGES_TOTAL = 160000
NPAD = 10240
DB = 128
NS = 16
NC = 2
RPT = NPAD // NS
EPT = N_EDGES_TOTAL // NS
CH = 50
NJ = 10
NG = EPT // (NJ * CH)
BN = 1000


@functools.lru_cache(maxsize=None)
def _make_segsum(nb, with_counts):
    nbpc = nb // NC
    mesh = plsc.VectorSubcoreMesh(core_axis_name="c", subcore_axis_name="s")

    out_type = [jax.ShapeDtypeStruct((nb * NPAD, DB), jnp.float32)]
    if with_counts:
        out_type.append(jax.ShapeDtypeStruct((NPAD,), jnp.float32))

    scratch_types = [
        pltpu.VMEM((2, NJ, CH), jnp.int32),
        pltpu.VMEM((CH, DB), jnp.float32),
        pltpu.VMEM((CH, DB), jnp.float32),
        pltpu.VMEM((64,), jnp.float32),
        pltpu.VMEM_SHARED((NPAD, DB), jnp.float32),
        pltpu.VMEM_SHARED((NPAD,), jnp.float32),
        pltpu.SemaphoreType.DMA,
        pltpu.SemaphoreType.DMA,
        pltpu.SemaphoreType.DMA,
        pltpu.SemaphoreType.DMA,
        pltpu.SemaphoreType.DMA,
    ]

    @functools.partial(pl.kernel, mesh=mesh, out_type=tuple(out_type),
                       scratch_types=scratch_types)
    def segsum(tab, idxh, *refs):
        if with_counts:
            out, cnt_out = refs[0], refs[1]
            scratch = refs[2:]
        else:
            out = refs[0]
            scratch = refs[1:]
        (idxbuf, rows0, rows1, ones_v, acc, cnt_acc,
         gsem0, gsem1, ssem0, ssem1, csem) = scratch
        bufs = (rows0, rows1)
        gsems = (gsem0, gsem1)
        ssems = (ssem0, ssem1)

        c = lax.axis_index("c")
        s = lax.axis_index("s")
        row0 = s * RPT

        for kk in range(4):
            ones_v[pl.ds(kk * 16, 16)] = jnp.ones((16,), jnp.float32)

        def zero_acc():
            def zrow(r, carry):
                for kk in range(DB // 16):
                    rows0[r, pl.ds(kk * 16, 16)] = jnp.zeros((16,), jnp.float32)
                return carry
            lax.fori_loop(0, CH, zrow, 0)
            for j in range(RPT // 40):
                pltpu.sync_copy(rows0.at[pl.ds(0, 40)],
                                acc.at[pl.ds(row0 + j * 40, 40)])

        def zero_cnt():
            for j in range(RPT // DB):
                pltpu.sync_copy(rows0.at[0], cnt_acc.at[pl.ds(row0 + j * DB, DB)])

        zero_acc()
        if with_counts:
            @pl.when(c == 0)
            def _():
                zero_cnt()

        for bb in range(nbpc):
            if bb > 0:
                zero_acc()
            plsc.subcore_barrier()

            b = c * nbpc + bb
            do_counts = with_counts and bb == 0

            def gbody(g, carry):
                pltpu.sync_copy(idxh.at[b, s, g], idxbuf)
                gh = [None, None]
                sh = [None, None]
                chs = []
                gh[0] = pltpu.async_copy(tab.at[idxbuf.at[0, 0]], rows0, gsem0)
                gh[1] = pltpu.async_copy(tab.at[idxbuf.at[0, 1]], rows1, gsem1)
                for j in range(NJ):
                    bi = j % 2
                    gh[bi].wait()
                    sh[bi] = pltpu.async_copy(bufs[bi],
                                              acc.at[idxbuf.at[1, j]],
                                              ssems[bi], add=True)
                    if do_counts:
                        @pl.when(c == 0)
                        def _():
                            chs.append(pltpu.async_copy(
                                ones_v.at[pl.ds(0, CH)],
                                cnt_acc.at[idxbuf.at[1, j]], csem, add=True))
                    if j + 2 < NJ:
                        sh[bi].wait()
                        gh[bi] = pltpu.async_copy(tab.at[idxbuf.at[0, j + 2]],
                                                  bufs[bi], gsems[bi])
                sh[(NJ - 2) % 2].wait()
                sh[(NJ - 1) % 2].wait()
                if do_counts:
                    @pl.when(c == 0)
                    def _():
                        for ch in chs:
                            ch.wait()
                return carry
            lax.fori_loop(0, NG, gbody, 0)

            plsc.subcore_barrier()

            brow = b * NPAD + row0
            for j in range(RPT // DB):
                pltpu.sync_copy(acc.at[pl.ds(row0 + j * DB, DB)],
                                out.at[pl.ds(brow + j * DB, DB)])
            if with_counts and bb == 0:
                @pl.when(c == 0)
                def _():
                    pltpu.sync_copy(cnt_acc.at[pl.ds(row0, RPT)],
                                    cnt_out.at[pl.ds(row0, RPT)])
            if bb + 1 < nbpc:
                plsc.subcore_barrier()

    return segsum


def _aggregate(table, idxb, nb, with_counts):
    res = _make_segsum(nb, with_counts)(table, idxb)
    if with_counts:
        aggflat, cnt = res
    else:
        aggflat = res[0] if isinstance(res, (tuple, list)) else res
    agg = aggflat.reshape(nb, NPAD, DB)[:, :N_NODES]
    agg = agg.transpose(1, 0, 2).reshape(N_NODES, nb * DB)
    if with_counts:
        return agg, cnt[:N_NODES, None]
    return agg


def _matmul_body(a_ref, wt_ref, o_ref):
    o_ref[...] = jnp.dot(a_ref[...], wt_ref[...],
                         preferred_element_type=jnp.float32)


def _dense_matmul(a, wT):
    din, h = wT.shape
    return pl.pallas_call(
        _matmul_body,
        grid=(N_NODES // BN,),
        in_specs=[
            pl.BlockSpec((BN, din), lambda i: (i, 0)),
            pl.BlockSpec((din, h), lambda i: (0, 0)),
        ],
        out_specs=pl.BlockSpec((BN, h), lambda i: (i, 0)),
        out_shape=jax.ShapeDtypeStruct((N_NODES, h), jnp.float32),
    )(a, wT)


def _combine1_body(cnt_ref, agg_ref, xr_ref, wlt_ref, bl_ref, o_ref):
    inv = 1.0 / jnp.clip(cnt_ref[...], 1.0, None)
    mean = agg_ref[...] * inv
    h = jnp.dot(mean, wlt_ref[...], preferred_element_type=jnp.float32)
    o_ref[...] = jnp.maximum(h + xr_ref[...] + bl_ref[...], 0.0)


def _combine1(cnt, agg, xr, WlT, bl):
    din, h = WlT.shape
    return pl.pallas_call(
        _combine1_body,
        grid=(N_NODES // BN,),
        in_specs=[
            pl.BlockSpec((BN, 1), lambda i: (i, 0)),
            pl.BlockSpec((BN, din), lambda i: (i, 0)),
            pl.BlockSpec((BN, h), lambda i: (i, 0)),
            pl.BlockSpec((din, h), lambda i: (0, 0)),
            pl.BlockSpec((1, h), lambda i: (0, 0)),
        ],
        out_specs=pl.BlockSpec((BN, h), lambda i: (i, 0)),
        out_shape=jax.ShapeDtypeStruct((N_NODES, h), jnp.float32),
    )(cnt, agg, xr, WlT, bl)


def _combine2_body(cnt_ref, agg_ref, xr_ref, wlt_ref, bl_ref,
                   wlint_ref, blin_ref, o_ref):
    inv = 1.0 / jnp.clip(cnt_ref[...], 1.0, None)
    mean = agg_ref[...] * inv
    h2 = jnp.dot(mean, wlt_ref[...], preferred_element_type=jnp.float32)
    h2 = jnp.maximum(h2 + xr_ref[...] + bl_ref[...], 0.0)
    out = jnp.dot(h2, wlint_ref[...], preferred_element_type=jnp.float32)
    o_ref[...] = out + blin_ref[...]


def _combine2(cnt, agg, xr, Wl2T, bl2, WlinT, blin):
    din, h = Wl2T.shape
    out = WlinT.shape[1]
    return pl.pallas_call(
        _combine2_body,
        grid=(N_NODES // BN,),
        in_specs=[
            pl.BlockSpec((BN, 1), lambda i: (i, 0)),
            pl.BlockSpec((BN, din), lambda i: (i, 0)),
            pl.BlockSpec((BN, h), lambda i: (i, 0)),
            pl.BlockSpec((din, h), lambda i: (0, 0)),
            pl.BlockSpec((1, h), lambda i: (0, 0)),
            pl.BlockSpec((h, out), lambda i: (0, 0)),
            pl.BlockSpec((1, out), lambda i: (0, 0)),
        ],
        out_specs=pl.BlockSpec((BN, out), lambda i: (i, 0)),
        out_shape=jax.ShapeDtypeStruct((N_NODES, out), jnp.float32),
    )(cnt, agg, xr, Wl2T, bl2, WlinT, blin)


def kernel(x, edge_index, Wl1, bl1, Wr1, Wl2, bl2, Wr2, Wlin, blin):
    src = edge_index[0].astype(jnp.int32)
    dst = edge_index[1].astype(jnp.int32)

    def idx_blocks(nb):
        offs = (jnp.arange(nb, dtype=jnp.int32) * N_NODES)[:, None]
        srcb = src[None, :] + offs
        dstb = jnp.broadcast_to(dst[None, :], (nb, N_EDGES_TOTAL))
        comb = jnp.stack([srcb, dstb], axis=1)
        comb = comb.reshape(nb, 2, NS, NG, NJ, CH)
        return comb.transpose(0, 2, 3, 1, 4, 5)

    nb1 = x.shape[1] // DB
    xt = x.reshape(N_NODES, nb1, DB).transpose(1, 0, 2).reshape(nb1 * N_NODES, DB)
    xr1 = _dense_matmul(x, Wr1.T)
    agg1, cnt = _aggregate(xt, idx_blocks(nb1), nb1, True)
    h1 = _combine1(cnt, agg1, xr1, Wl1.T, bl1[None, :])

    nb2 = h1.shape[1] // DB
    ht = h1.reshape(N_NODES, nb2, DB).transpose(1, 0, 2).reshape(nb2 * N_NODES, DB)
    xr2 = _dense_matmul(h1, Wr2.T)
    agg2 = _aggregate(ht, idx_blocks(nb2), nb2, False)
    return _combine2(cnt, agg2, xr2, Wl2.T, bl2[None, :], Wlin.T, blin[None, :])


# speedup vs baseline: 5.3134x; 1.0873x over previous
import functools

import jax
import jax.numpy as jnp
from jax import lax
from jax.experimental import pallas as pl
from jax.experimental.pallas import tpu as pltpu
from jax.experimental.pallas import tpu_sc as plsc

N_NODES = 10000
N_EDGES_TOTAL = 160000
NPAD = 10240
DB = 128
NS = 16
NC = 2
RPT = NPAD // NS
EPT = N_EDGES_TOTAL // NS
CH = 50
NJ = 10
NG = EPT // (NJ * CH)
BN = 1000


@functools.lru_cache(maxsize=None)
def _make_segsum(nb, with_counts):
    nbpc = nb // NC
    mesh = plsc.VectorSubcoreMesh(core_axis_name="c", subcore_axis_name="s")

    out_type = [jax.ShapeDtypeStruct((nb * NPAD, DB), jnp.float32)]
    if with_counts:
        out_type.append(jax.ShapeDtypeStruct((NPAD,), jnp.float32))

    scratch_types = [
        pltpu.VMEM((2, NJ, CH), jnp.int32),
        pltpu.VMEM((CH, DB), jnp.float32),
        pltpu.VMEM((CH, DB), jnp.float32),
        pltpu.VMEM((CH, DB), jnp.float32),
        pltpu.VMEM((64,), jnp.float32),
        pltpu.VMEM_SHARED((NPAD, DB), jnp.float32),
        pltpu.VMEM_SHARED((NPAD,), jnp.float32),
        pltpu.SemaphoreType.DMA,
        pltpu.SemaphoreType.DMA,
        pltpu.SemaphoreType.DMA,
        pltpu.SemaphoreType.DMA,
        pltpu.SemaphoreType.DMA,
        pltpu.SemaphoreType.DMA,
        pltpu.SemaphoreType.DMA,
    ]

    @functools.partial(pl.kernel, mesh=mesh, out_type=tuple(out_type),
                       scratch_types=scratch_types)
    def segsum(tab, idxh, *refs):
        if with_counts:
            out, cnt_out = refs[0], refs[1]
            scratch = refs[2:]
        else:
            out = refs[0]
            scratch = refs[1:]
        (idxbuf, rows0, rows1, rows2, ones_v, acc, cnt_acc,
         gsem0, gsem1, gsem2, ssem0, ssem1, ssem2, csem) = scratch
        bufs = (rows0, rows1, rows2)
        gsems = (gsem0, gsem1, gsem2)
        ssems = (ssem0, ssem1, ssem2)

        c = lax.axis_index("c")
        s = lax.axis_index("s")
        row0 = s * RPT

        for kk in range(4):
            ones_v[pl.ds(kk * 16, 16)] = jnp.ones((16,), jnp.float32)

        def zero_acc():
            def zrow(r, carry):
                for kk in range(DB // 16):
                    rows0[r, pl.ds(kk * 16, 16)] = jnp.zeros((16,), jnp.float32)
                return carry
            lax.fori_loop(0, CH, zrow, 0)
            for j in range(RPT // 40):
                pltpu.sync_copy(rows0.at[pl.ds(0, 40)],
                                acc.at[pl.ds(row0 + j * 40, 40)])

        def zero_cnt():
            for j in range(RPT // DB):
                pltpu.sync_copy(rows0.at[0], cnt_acc.at[pl.ds(row0 + j * DB, DB)])

        zero_acc()
        if with_counts:
            @pl.when(c == 0)
            def _():
                zero_cnt()

        for bb in range(nbpc):
            if bb > 0:
                zero_acc()
            plsc.subcore_barrier()

            b = c * nbpc + bb
            do_counts = with_counts and bb == 0

            def gbody(g, carry):
                pltpu.sync_copy(idxh.at[b, s, g], idxbuf)
                gh = [None, None, None]
                sh = [None, None, None]
                chs = []
                for jj in range(3):
                    gh[jj] = pltpu.async_copy(tab.at[idxbuf.at[0, jj]],
                                              bufs[jj], gsems[jj])
                for j in range(NJ):
                    bi = j % 3
                    gh[bi].wait()
                    sh[bi] = pltpu.async_copy(bufs[bi],
                                              acc.at[idxbuf.at[1, j]],
                                              ssems[bi], add=True)
                    if do_counts:
                        @pl.when(c == 0)
                        def _():
                            chs.append(pltpu.async_copy(
                                ones_v.at[pl.ds(0, CH)],
                                cnt_acc.at[idxbuf.at[1, j]], csem, add=True))
                    if j + 3 < NJ:
                        sh[bi].wait()
                        gh[bi] = pltpu.async_copy(tab.at[idxbuf.at[0, j + 3]],
                                                  bufs[bi], gsems[bi])
                sh[(NJ - 3) % 3].wait()
                sh[(NJ - 2) % 3].wait()
                sh[(NJ - 1) % 3].wait()
                if do_counts:
                    @pl.when(c == 0)
                    def _():
                        for ch in chs:
                            ch.wait()
                return carry
            lax.fori_loop(0, NG, gbody, 0)

            plsc.subcore_barrier()

            brow = b * NPAD + row0
            for j in range(RPT // DB):
                pltpu.sync_copy(acc.at[pl.ds(row0 + j * DB, DB)],
                                out.at[pl.ds(brow + j * DB, DB)])
            if with_counts and bb == 0:
                @pl.when(c == 0)
                def _():
                    pltpu.sync_copy(cnt_acc.at[pl.ds(row0, RPT)],
                                    cnt_out.at[pl.ds(row0, RPT)])
            if bb + 1 < nbpc:
                plsc.subcore_barrier()

    return segsum


def _aggregate(table, idxb, nb, with_counts):
    res = _make_segsum(nb, with_counts)(table, idxb)
    if with_counts:
        aggflat, cnt = res
    else:
        aggflat = res[0] if isinstance(res, (tuple, list)) else res
    agg = aggflat.reshape(nb, NPAD, DB)[:, :N_NODES]
    agg = agg.transpose(1, 0, 2).reshape(N_NODES, nb * DB)
    if with_counts:
        return agg, cnt[:N_NODES, None]
    return agg


def _matmul_body(a_ref, wt_ref, o_ref):
    o_ref[...] = jnp.dot(a_ref[...], wt_ref[...],
                         preferred_element_type=jnp.float32)


def _dense_matmul(a, wT):
    din, h = wT.shape
    return pl.pallas_call(
        _matmul_body,
        grid=(N_NODES // BN,),
        in_specs=[
            pl.BlockSpec((BN, din), lambda i: (i, 0)),
            pl.BlockSpec((din, h), lambda i: (0, 0)),
        ],
        out_specs=pl.BlockSpec((BN, h), lambda i: (i, 0)),
        out_shape=jax.ShapeDtypeStruct((N_NODES, h), jnp.float32),
    )(a, wT)


def _combine1_body(cnt_ref, agg_ref, xr_ref, wlt_ref, bl_ref, o_ref):
    inv = 1.0 / jnp.clip(cnt_ref[...], 1.0, None)
    mean = agg_ref[...] * inv
    h = jnp.dot(mean, wlt_ref[...], preferred_element_type=jnp.float32)
    o_ref[...] = jnp.maximum(h + xr_ref[...] + bl_ref[...], 0.0)


def _combine1(cnt, agg, xr, WlT, bl):
    din, h = WlT.shape
    return pl.pallas_call(
        _combine1_body,
        grid=(N_NODES // BN,),
        in_specs=[
            pl.BlockSpec((BN, 1), lambda i: (i, 0)),
            pl.BlockSpec((BN, din), lambda i: (i, 0)),
            pl.BlockSpec((BN, h), lambda i: (i, 0)),
            pl.BlockSpec((din, h), lambda i: (0, 0)),
            pl.BlockSpec((1, h), lambda i: (0, 0)),
        ],
        out_specs=pl.BlockSpec((BN, h), lambda i: (i, 0)),
        out_shape=jax.ShapeDtypeStruct((N_NODES, h), jnp.float32),
    )(cnt, agg, xr, WlT, bl)


def _combine2_body(cnt_ref, agg_ref, xr_ref, wlt_ref, bl_ref,
                   wlint_ref, blin_ref, o_ref):
    inv = 1.0 / jnp.clip(cnt_ref[...], 1.0, None)
    mean = agg_ref[...] * inv
    h2 = jnp.dot(mean, wlt_ref[...], preferred_element_type=jnp.float32)
    h2 = jnp.maximum(h2 + xr_ref[...] + bl_ref[...], 0.0)
    out = jnp.dot(h2, wlint_ref[...], preferred_element_type=jnp.float32)
    o_ref[...] = out + blin_ref[...]


def _combine2(cnt, agg, xr, Wl2T, bl2, WlinT, blin):
    din, h = Wl2T.shape
    out = WlinT.shape[1]
    return pl.pallas_call(
        _combine2_body,
        grid=(N_NODES // BN,),
        in_specs=[
            pl.BlockSpec((BN, 1), lambda i: (i, 0)),
            pl.BlockSpec((BN, din), lambda i: (i, 0)),
            pl.BlockSpec((BN, h), lambda i: (i, 0)),
            pl.BlockSpec((din, h), lambda i: (0, 0)),
            pl.BlockSpec((1, h), lambda i: (0, 0)),
            pl.BlockSpec((h, out), lambda i: (0, 0)),
            pl.BlockSpec((1, out), lambda i: (0, 0)),
        ],
        out_specs=pl.BlockSpec((BN, out), lambda i: (i, 0)),
        out_shape=jax.ShapeDtypeStruct((N_NODES, out), jnp.float32),
    )(cnt, agg, xr, Wl2T, bl2, WlinT, blin)


def kernel(x, edge_index, Wl1, bl1, Wr1, Wl2, bl2, Wr2, Wlin, blin):
    src = edge_index[0].astype(jnp.int32)
    dst = edge_index[1].astype(jnp.int32)

    def idx_blocks(nb):
        offs = jnp.arange(nb, dtype=jnp.int32)[:, None]
        srcb = src[None, :] * nb + offs
        dstb = jnp.broadcast_to(dst[None, :], (nb, N_EDGES_TOTAL))
        comb = jnp.stack([srcb, dstb], axis=1)
        comb = comb.reshape(nb, 2, NS, NG, NJ, CH)
        return comb.transpose(0, 2, 3, 1, 4, 5)

    nb1 = x.shape[1] // DB
    xt = x.reshape(nb1 * N_NODES, DB)
    xr1 = _dense_matmul(x, Wr1.T)
    agg1, cnt = _aggregate(xt, idx_blocks(nb1), nb1, True)
    h1 = _combine1(cnt, agg1, xr1, Wl1.T, bl1[None, :])

    nb2 = h1.shape[1] // DB
    ht = h1.reshape(nb2 * N_NODES, DB)
    xr2 = _dense_matmul(h1, Wr2.T)
    agg2 = _aggregate(ht, idx_blocks(nb2), nb2, False)
    return _combine2(cnt, agg2, xr2, Wl2.T, bl2[None, :], Wlin.T, blin[None, :])


# speedup vs baseline: 6.2189x; 1.1704x over previous
import functools

import jax
import jax.numpy as jnp
from jax import lax
from jax.experimental import pallas as pl
from jax.experimental.pallas import tpu as pltpu
from jax.experimental.pallas import tpu_sc as plsc

N_NODES = 10000
N_EDGES_TOTAL = 160000
NPAD = 10240
DB = 128
NS = 16
NC = 2
RPT = NPAD // NS
EPT = N_EDGES_TOTAL // NS
CH = 50
NJ = 10
NG = EPT // (NJ * CH)
BN = 1000


@functools.lru_cache(maxsize=None)
def _make_segsum(nb, with_counts):
    nbpc = nb // NC
    mesh = plsc.VectorSubcoreMesh(core_axis_name="c", subcore_axis_name="s")

    out_type = [jax.ShapeDtypeStruct((NPAD, nb * DB), jnp.float32)]
    if with_counts:
        out_type.append(jax.ShapeDtypeStruct((NPAD,), jnp.float32))

    scratch_types = [
        pltpu.VMEM((2, NJ, CH), jnp.int32),
        pltpu.VMEM((CH, DB), jnp.float32),
        pltpu.VMEM((CH, DB), jnp.float32),
        pltpu.VMEM((CH, DB), jnp.float32),
        pltpu.VMEM((64,), jnp.float32),
        pltpu.VMEM_SHARED((NPAD, DB), jnp.float32),
        pltpu.VMEM_SHARED((NPAD,), jnp.float32),
        pltpu.SemaphoreType.DMA,
        pltpu.SemaphoreType.DMA,
        pltpu.SemaphoreType.DMA,
        pltpu.SemaphoreType.DMA,
        pltpu.SemaphoreType.DMA,
        pltpu.SemaphoreType.DMA,
        pltpu.SemaphoreType.DMA,
    ]

    @functools.partial(pl.kernel, mesh=mesh, out_type=tuple(out_type),
                       scratch_types=scratch_types)
    def segsum(tab, idxh, *refs):
        if with_counts:
            out, cnt_out = refs[0], refs[1]
            scratch = refs[2:]
        else:
            out = refs[0]
            scratch = refs[1:]
        (idxbuf, rows0, rows1, rows2, ones_v, acc, cnt_acc,
         gsem0, gsem1, gsem2, ssem0, ssem1, ssem2, csem) = scratch
        bufs = (rows0, rows1, rows2)
        gsems = (gsem0, gsem1, gsem2)
        ssems = (ssem0, ssem1, ssem2)

        c = lax.axis_index("c")
        s = lax.axis_index("s")
        row0 = s * RPT

        for kk in range(4):
            ones_v[pl.ds(kk * 16, 16)] = jnp.ones((16,), jnp.float32)

        def zero_acc():
            def zrow(r, carry):
                for kk in range(DB // 16):
                    rows0[r, pl.ds(kk * 16, 16)] = jnp.zeros((16,), jnp.float32)
                return carry
            lax.fori_loop(0, CH, zrow, 0)
            for j in range(RPT // 40):
                pltpu.sync_copy(rows0.at[pl.ds(0, 40)],
                                acc.at[pl.ds(row0 + j * 40, 40)])

        def zero_cnt():
            for j in range(RPT // DB):
                pltpu.sync_copy(rows0.at[0], cnt_acc.at[pl.ds(row0 + j * DB, DB)])

        zero_acc()
        if with_counts:
            @pl.when(c == 0)
            def _():
                zero_cnt()

        for bb in range(nbpc):
            if bb > 0:
                zero_acc()
            plsc.subcore_barrier()

            b = c * nbpc + bb
            do_counts = with_counts and bb == 0

            def gbody(g, carry):
                pltpu.sync_copy(idxh.at[b, s, g], idxbuf)
                gh = [None, None, None]
                sh = [None, None, None]
                chs = []
                for jj in range(3):
                    gh[jj] = pltpu.async_copy(tab.at[idxbuf.at[0, jj]],
                                              bufs[jj], gsems[jj])
                for j in range(NJ):
                    bi = j % 3
                    gh[bi].wait()
                    sh[bi] = pltpu.async_copy(bufs[bi],
                                              acc.at[idxbuf.at[1, j]],
                                              ssems[bi], add=True)
                    if do_counts:
                        @pl.when(c == 0)
                        def _():
                            chs.append(pltpu.async_copy(
                                ones_v.at[pl.ds(0, CH)],
                                cnt_acc.at[idxbuf.at[1, j]], csem, add=True))
                    if j + 3 < NJ:
                        sh[bi].wait()
                        gh[bi] = pltpu.async_copy(tab.at[idxbuf.at[0, j + 3]],
                                                  bufs[bi], gsems[bi])
                sh[(NJ - 3) % 3].wait()
                sh[(NJ - 2) % 3].wait()
                sh[(NJ - 1) % 3].wait()
                if do_counts:
                    @pl.when(c == 0)
                    def _():
                        for ch in chs:
                            ch.wait()
                return carry
            lax.fori_loop(0, NG, gbody, 0)

            plsc.subcore_barrier()

            bcol = b * DB
            for j in range(RPT // DB):
                pltpu.sync_copy(acc.at[pl.ds(row0 + j * DB, DB)],
                                out.at[pl.ds(row0 + j * DB, DB),
                                       pl.ds(bcol, DB)])
            if with_counts and bb == 0:
                @pl.when(c == 0)
                def _():
                    pltpu.sync_copy(cnt_acc.at[pl.ds(row0, RPT)],
                                    cnt_out.at[pl.ds(row0, RPT)])
            if bb + 1 < nbpc:
                plsc.subcore_barrier()

    return segsum


def _aggregate(table, idxb, nb, with_counts):
    res = _make_segsum(nb, with_counts)(table, idxb)
    if with_counts:
        agg, cnt = res
        return agg, cnt[:, None]
    return res[0] if isinstance(res, (tuple, list)) else res


def _matmul_body(a_ref, wt_ref, o_ref):
    o_ref[...] = jnp.dot(a_ref[...], wt_ref[...],
                         preferred_element_type=jnp.float32)


def _dense_matmul(a, wT):
    din, h = wT.shape
    return pl.pallas_call(
        _matmul_body,
        grid=(N_NODES // BN,),
        in_specs=[
            pl.BlockSpec((BN, din), lambda i: (i, 0)),
            pl.BlockSpec((din, h), lambda i: (0, 0)),
        ],
        out_specs=pl.BlockSpec((BN, h), lambda i: (i, 0)),
        out_shape=jax.ShapeDtypeStruct((N_NODES, h), jnp.float32),
    )(a, wT)


def _combine1_body(cnt_ref, agg_ref, xr_ref, wlt_ref, bl_ref, o_ref):
    inv = 1.0 / jnp.clip(cnt_ref[...], 1.0, None)
    mean = agg_ref[...] * inv
    h = jnp.dot(mean, wlt_ref[...], preferred_element_type=jnp.float32)
    o_ref[...] = jnp.maximum(h + xr_ref[...] + bl_ref[...], 0.0)


def _combine1(cnt, agg, xr, WlT, bl):
    din, h = WlT.shape
    return pl.pallas_call(
        _combine1_body,
        grid=(N_NODES // BN,),
        in_specs=[
            pl.BlockSpec((BN, 1), lambda i: (i, 0)),
            pl.BlockSpec((BN, din), lambda i: (i, 0)),
            pl.BlockSpec((BN, h), lambda i: (i, 0)),
            pl.BlockSpec((din, h), lambda i: (0, 0)),
            pl.BlockSpec((1, h), lambda i: (0, 0)),
        ],
        out_specs=pl.BlockSpec((BN, h), lambda i: (i, 0)),
        out_shape=jax.ShapeDtypeStruct((N_NODES, h), jnp.float32),
    )(cnt, agg, xr, WlT, bl)


def _combine2_body(cnt_ref, agg_ref, xr_ref, wlt_ref, bl_ref,
                   wlint_ref, blin_ref, o_ref):
    inv = 1.0 / jnp.clip(cnt_ref[...], 1.0, None)
    mean = agg_ref[...] * inv
    h2 = jnp.dot(mean, wlt_ref[...], preferred_element_type=jnp.float32)
    h2 = jnp.maximum(h2 + xr_ref[...] + bl_ref[...], 0.0)
    out = jnp.dot(h2, wlint_ref[...], preferred_element_type=jnp.float32)
    o_ref[...] = out + blin_ref[...]


def _combine2(cnt, agg, xr, Wl2T, bl2, WlinT, blin):
    din, h = Wl2T.shape
    out = WlinT.shape[1]
    return pl.pallas_call(
        _combine2_body,
        grid=(N_NODES // BN,),
        in_specs=[
            pl.BlockSpec((BN, 1), lambda i: (i, 0)),
            pl.BlockSpec((BN, din), lambda i: (i, 0)),
            pl.BlockSpec((BN, h), lambda i: (i, 0)),
            pl.BlockSpec((din, h), lambda i: (0, 0)),
            pl.BlockSpec((1, h), lambda i: (0, 0)),
            pl.BlockSpec((h, out), lambda i: (0, 0)),
            pl.BlockSpec((1, out), lambda i: (0, 0)),
        ],
        out_specs=pl.BlockSpec((BN, out), lambda i: (i, 0)),
        out_shape=jax.ShapeDtypeStruct((N_NODES, out), jnp.float32),
    )(cnt, agg, xr, Wl2T, bl2, WlinT, blin)


def kernel(x, edge_index, Wl1, bl1, Wr1, Wl2, bl2, Wr2, Wlin, blin):
    src = edge_index[0].astype(jnp.int32)
    dst = edge_index[1].astype(jnp.int32)

    def idx_blocks(nb):
        offs = jnp.arange(nb, dtype=jnp.int32)[:, None]
        srcb = src[None, :] * nb + offs
        dstb = jnp.broadcast_to(dst[None, :], (nb, N_EDGES_TOTAL))
        comb = jnp.stack([srcb, dstb], axis=1)
        comb = comb.reshape(nb, 2, NS, NG, NJ, CH)
        return comb.transpose(0, 2, 3, 1, 4, 5)

    nb1 = x.shape[1] // DB
    xt = x.reshape(nb1 * N_NODES, DB)
    xr1 = _dense_matmul(x, Wr1.T)
    agg1, cnt = _aggregate(xt, idx_blocks(nb1), nb1, True)
    h1 = _combine1(cnt, agg1, xr1, Wl1.T, bl1[None, :])

    nb2 = h1.shape[1] // DB
    ht = h1.reshape(nb2 * N_NODES, DB)
    xr2 = _dense_matmul(h1, Wr2.T)
    agg2 = _aggregate(ht, idx_blocks(nb2), nb2, False)
    return _combine2(cnt, agg2, xr2, Wl2.T, bl2[None, :], Wlin.T, blin[None, :])


# speedup vs baseline: 6.6259x; 1.0654x over previous
import functools

import jax
import jax.numpy as jnp
from jax import lax
from jax.experimental import pallas as pl
from jax.experimental.pallas import tpu as pltpu
from jax.experimental.pallas import tpu_sc as plsc

N_NODES = 10000
N_EDGES_TOTAL = 160000
NPAD = 10240
DB = 128
NS = 16
NC = 2
RPT = NPAD // NS
EPT = N_EDGES_TOTAL // NS
CH = 50
NJ = 10
NG = EPT // (NJ * CH)
BN = 1000


@functools.lru_cache(maxsize=None)
def _make_segsum(nb, with_counts):
    nbpc = nb // NC
    mesh = plsc.VectorSubcoreMesh(core_axis_name="c", subcore_axis_name="s")

    out_type = [jax.ShapeDtypeStruct((NPAD, nb * DB), jnp.float32)]
    if with_counts:
        out_type.append(jax.ShapeDtypeStruct((NPAD,), jnp.float32))

    scratch_types = [
        pltpu.VMEM((2, 2, NJ, CH), jnp.int32),
        pltpu.VMEM((CH, DB), jnp.float32),
        pltpu.VMEM((CH, DB), jnp.float32),
        pltpu.VMEM((CH, DB), jnp.float32),
        pltpu.VMEM((64,), jnp.float32),
        pltpu.VMEM_SHARED((NPAD, DB), jnp.float32),
        pltpu.VMEM_SHARED((NPAD,), jnp.float32),
        pltpu.SemaphoreType.DMA,
        pltpu.SemaphoreType.DMA,
        pltpu.SemaphoreType.DMA,
        pltpu.SemaphoreType.DMA,
        pltpu.SemaphoreType.DMA,
        pltpu.SemaphoreType.DMA,
        pltpu.SemaphoreType.DMA,
        pltpu.SemaphoreType.DMA,
    ]

    @functools.partial(pl.kernel, mesh=mesh, out_type=tuple(out_type),
                       scratch_types=scratch_types)
    def segsum(tab, idxh, *refs):
        if with_counts:
            out, cnt_out = refs[0], refs[1]
            scratch = refs[2:]
        else:
            out = refs[0]
            scratch = refs[1:]
        (idxbuf, rows0, rows1, rows2, ones_v, acc, cnt_acc,
         gsem0, gsem1, gsem2, ssem0, ssem1, ssem2, csem, isem) = scratch
        bufs = (rows0, rows1, rows2)
        gsems = (gsem0, gsem1, gsem2)
        ssems = (ssem0, ssem1, ssem2)

        c = lax.axis_index("c")
        s = lax.axis_index("s")
        row0 = s * RPT

        for kk in range(4):
            ones_v[pl.ds(kk * 16, 16)] = jnp.ones((16,), jnp.float32)

        def zero_acc():
            def zrow(r, carry):
                for kk in range(DB // 16):
                    rows0[r, pl.ds(kk * 16, 16)] = jnp.zeros((16,), jnp.float32)
                return carry
            lax.fori_loop(0, CH, zrow, 0)
            for j in range(RPT // 40):
                pltpu.sync_copy(rows0.at[pl.ds(0, 40)],
                                acc.at[pl.ds(row0 + j * 40, 40)])

        def zero_cnt():
            for j in range(RPT // DB):
                pltpu.sync_copy(rows0.at[0], cnt_acc.at[pl.ds(row0 + j * DB, DB)])

        zero_acc()
        if with_counts:
            @pl.when(c == 0)
            def _():
                zero_cnt()

        for bb in range(nbpc):
            if bb > 0:
                zero_acc()
            plsc.subcore_barrier()

            b = c * nbpc + bb
            do_counts = with_counts and bb == 0

            pltpu.sync_copy(idxh.at[b, s, 0], idxbuf.at[0])

            def gbody(g, carry):
                p = lax.rem(g, 2)
                gg = jnp.minimum(g + 1, NG - 1)
                hpre = pltpu.async_copy(idxh.at[b, s, gg],
                                        idxbuf.at[1 - p], isem)
                cur = idxbuf.at[p]
                gh = [None, None, None]
                sh = [None, None, None]
                chs = []
                for jj in range(3):
                    gh[jj] = pltpu.async_copy(tab.at[cur.at[0, jj]],
                                              bufs[jj], gsems[jj])
                for j in range(NJ):
                    bi = j % 3
                    gh[bi].wait()
                    sh[bi] = pltpu.async_copy(bufs[bi],
                                              acc.at[cur.at[1, j]],
                                              ssems[bi], add=True)
                    if do_counts:
                        @pl.when(c == 0)
                        def _():
                            chs.append(pltpu.async_copy(
                                ones_v.at[pl.ds(0, CH)],
                                cnt_acc.at[cur.at[1, j]], csem, add=True))
                    if j + 3 < NJ:
                        sh[bi].wait()
                        gh[bi] = pltpu.async_copy(tab.at[cur.at[0, j + 3]],
                                                  bufs[bi], gsems[bi])
                sh[(NJ - 3) % 3].wait()
                sh[(NJ - 2) % 3].wait()
                sh[(NJ - 1) % 3].wait()
                if do_counts:
                    @pl.when(c == 0)
                    def _():
                        for ch in chs:
                            ch.wait()
                hpre.wait()
                return carry
            lax.fori_loop(0, NG, gbody, 0)

            plsc.subcore_barrier()

            bcol = b * DB
            for j in range(RPT // DB):
                pltpu.sync_copy(acc.at[pl.ds(row0 + j * DB, DB)],
                                out.at[pl.ds(row0 + j * DB, DB),
                                       pl.ds(bcol, DB)])
            if with_counts and bb == 0:
                @pl.when(c == 0)
                def _():
                    pltpu.sync_copy(cnt_acc.at[pl.ds(row0, RPT)],
                                    cnt_out.at[pl.ds(row0, RPT)])
            if bb + 1 < nbpc:
                plsc.subcore_barrier()

    return segsum


def _aggregate(table, idxb, nb, with_counts):
    res = _make_segsum(nb, with_counts)(table, idxb)
    if with_counts:
        agg, cnt = res
        return agg, cnt[:, None]
    return res[0] if isinstance(res, (tuple, list)) else res


def _matmul_body(a_ref, wt_ref, o_ref):
    o_ref[...] = jnp.dot(a_ref[...], wt_ref[...],
                         preferred_element_type=jnp.float32)


def _dense_matmul(a, wT):
    din, h = wT.shape
    return pl.pallas_call(
        _matmul_body,
        grid=(N_NODES // BN,),
        in_specs=[
            pl.BlockSpec((BN, din), lambda i: (i, 0)),
            pl.BlockSpec((din, h), lambda i: (0, 0)),
        ],
        out_specs=pl.BlockSpec((BN, h), lambda i: (i, 0)),
        out_shape=jax.ShapeDtypeStruct((N_NODES, h), jnp.float32),
    )(a, wT)


def _combine1_body(cnt_ref, agg_ref, xr_ref, wlt_ref, bl_ref, o_ref):
    inv = 1.0 / jnp.clip(cnt_ref[...], 1.0, None)
    mean = agg_ref[...] * inv
    h = jnp.dot(mean, wlt_ref[...], preferred_element_type=jnp.float32)
    o_ref[...] = jnp.maximum(h + xr_ref[...] + bl_ref[...], 0.0)


def _combine1(cnt, agg, xr, WlT, bl):
    din, h = WlT.shape
    return pl.pallas_call(
        _combine1_body,
        grid=(N_NODES // BN,),
        in_specs=[
            pl.BlockSpec((BN, 1), lambda i: (i, 0)),
            pl.BlockSpec((BN, din), lambda i: (i, 0)),
            pl.BlockSpec((BN, h), lambda i: (i, 0)),
            pl.BlockSpec((din, h), lambda i: (0, 0)),
            pl.BlockSpec((1, h), lambda i: (0, 0)),
        ],
        out_specs=pl.BlockSpec((BN, h), lambda i: (i, 0)),
        out_shape=jax.ShapeDtypeStruct((N_NODES, h), jnp.float32),
    )(cnt, agg, xr, WlT, bl)


def _combine2_body(cnt_ref, agg_ref, xr_ref, wlt_ref, bl_ref,
                   wlint_ref, blin_ref, o_ref):
    inv = 1.0 / jnp.clip(cnt_ref[...], 1.0, None)
    mean = agg_ref[...] * inv
    h2 = jnp.dot(mean, wlt_ref[...], preferred_element_type=jnp.float32)
    h2 = jnp.maximum(h2 + xr_ref[...] + bl_ref[...], 0.0)
    out = jnp.dot(h2, wlint_ref[...], preferred_element_type=jnp.float32)
    o_ref[...] = out + blin_ref[...]


def _combine2(cnt, agg, xr, Wl2T, bl2, WlinT, blin):
    din, h = Wl2T.shape
    out = WlinT.shape[1]
    return pl.pallas_call(
        _combine2_body,
        grid=(N_NODES // BN,),
        in_specs=[
            pl.BlockSpec((BN, 1), lambda i: (i, 0)),
            pl.BlockSpec((BN, din), lambda i: (i, 0)),
            pl.BlockSpec((BN, h), lambda i: (i, 0)),
            pl.BlockSpec((din, h), lambda i: (0, 0)),
            pl.BlockSpec((1, h), lambda i: (0, 0)),
            pl.BlockSpec((h, out), lambda i: (0, 0)),
            pl.BlockSpec((1, out), lambda i: (0, 0)),
        ],
        out_specs=pl.BlockSpec((BN, out), lambda i: (i, 0)),
        out_shape=jax.ShapeDtypeStruct((N_NODES, out), jnp.float32),
    )(cnt, agg, xr, Wl2T, bl2, WlinT, blin)


def kernel(x, edge_index, Wl1, bl1, Wr1, Wl2, bl2, Wr2, Wlin, blin):
    src = edge_index[0].astype(jnp.int32)
    dst = edge_index[1].astype(jnp.int32)

    def idx_blocks(nb):
        offs = jnp.arange(nb, dtype=jnp.int32)[:, None]
        srcb = src[None, :] * nb + offs
        dstb = jnp.broadcast_to(dst[None, :], (nb, N_EDGES_TOTAL))
        comb = jnp.stack([srcb, dstb], axis=1)
        comb = comb.reshape(nb, 2, NS, NG, NJ, CH)
        return comb.transpose(0, 2, 3, 1, 4, 5)

    nb1 = x.shape[1] // DB
    xt = x.reshape(nb1 * N_NODES, DB)
    xr1 = _dense_matmul(x, Wr1.T)
    agg1, cnt = _aggregate(xt, idx_blocks(nb1), nb1, True)
    h1 = _combine1(cnt, agg1, xr1, Wl1.T, bl1[None, :])

    nb2 = h1.shape[1] // DB
    ht = h1.reshape(nb2 * N_NODES, DB)
    xr2 = _dense_matmul(h1, Wr2.T)
    agg2 = _aggregate(ht, idx_blocks(nb2), nb2, False)
    return _combine2(cnt, agg2, xr2, Wl2.T, bl2[None, :], Wlin.T, blin[None, :])


# speedup vs baseline: 7.1429x; 1.0780x over previous
import functools

import jax
import jax.numpy as jnp
from jax import lax
from jax.experimental import pallas as pl
from jax.experimental.pallas import tpu as pltpu
from jax.experimental.pallas import tpu_sc as plsc

N_NODES = 10000
N_EDGES_TOTAL = 160000
NPAD = 10240
DB = 128
NS = 16
NC = 2
RPT = NPAD // NS
EPT = N_EDGES_TOTAL // NS
CH = 100
NJ = 10
NG = EPT // (NJ * CH)
BN = 1000


@functools.lru_cache(maxsize=None)
def _make_segsum(nb, with_counts):
    nbpc = nb // NC
    mesh = plsc.VectorSubcoreMesh(core_axis_name="c", subcore_axis_name="s")

    out_type = [jax.ShapeDtypeStruct((NPAD, nb * DB), jnp.float32)]
    if with_counts:
        out_type.append(jax.ShapeDtypeStruct((NPAD,), jnp.float32))

    scratch_types = [
        pltpu.VMEM((2, 2, NJ, CH), jnp.int32),
        pltpu.VMEM((CH, DB), jnp.float32),
        pltpu.VMEM((CH, DB), jnp.float32),
        pltpu.VMEM((112,), jnp.float32),
        pltpu.VMEM_SHARED((NPAD, DB), jnp.float32),
        pltpu.VMEM_SHARED((NPAD,), jnp.float32),
        pltpu.SemaphoreType.DMA,
        pltpu.SemaphoreType.DMA,
        pltpu.SemaphoreType.DMA,
        pltpu.SemaphoreType.DMA,
        pltpu.SemaphoreType.DMA,
        pltpu.SemaphoreType.DMA,
    ]

    @functools.partial(pl.kernel, mesh=mesh, out_type=tuple(out_type),
                       scratch_types=scratch_types)
    def segsum(tab, idxh, *refs):
        if with_counts:
            out, cnt_out = refs[0], refs[1]
            scratch = refs[2:]
        else:
            out = refs[0]
            scratch = refs[1:]
        (idxbuf, rows0, rows1, ones_v, acc, cnt_acc,
         gsem0, gsem1, ssem0, ssem1, csem, isem) = scratch
        bufs = (rows0, rows1)
        gsems = (gsem0, gsem1)
        ssems = (ssem0, ssem1)

        c = lax.axis_index("c")
        s = lax.axis_index("s")
        row0 = s * RPT

        for kk in range(7):
            ones_v[pl.ds(kk * 16, 16)] = jnp.ones((16,), jnp.float32)

        def zero_acc():
            def zrow(r, carry):
                for kk in range(DB // 16):
                    rows0[r, pl.ds(kk * 16, 16)] = jnp.zeros((16,), jnp.float32)
                return carry
            lax.fori_loop(0, CH, zrow, 0)
            for j in range(RPT // 40):
                pltpu.sync_copy(rows0.at[pl.ds(0, 40)],
                                acc.at[pl.ds(row0 + j * 40, 40)])

        def zero_cnt():
            for j in range(RPT // DB):
                pltpu.sync_copy(rows0.at[0], cnt_acc.at[pl.ds(row0 + j * DB, DB)])

        zero_acc()
        if with_counts:
            @pl.when(c == 0)
            def _():
                zero_cnt()

        for bb in range(nbpc):
            if bb > 0:
                zero_acc()
            plsc.subcore_barrier()

            b = c * nbpc + bb
            do_counts = with_counts and bb == 0

            pltpu.sync_copy(idxh.at[b, s, 0], idxbuf.at[0])

            def gbody(g, carry):
                p = lax.rem(g, 2)
                gg = jnp.minimum(g + 1, NG - 1)
                hpre = pltpu.async_copy(idxh.at[b, s, gg],
                                        idxbuf.at[1 - p], isem)
                cur = idxbuf.at[p]
                gh = [None, None]
                sh = [None, None]
                chs = []
                for jj in range(2):
                    gh[jj] = pltpu.async_copy(tab.at[cur.at[0, jj]],
                                              bufs[jj], gsems[jj])
                for j in range(NJ):
                    bi = j % 2
                    gh[bi].wait()
                    sh[bi] = pltpu.async_copy(bufs[bi],
                                              acc.at[cur.at[1, j]],
                                              ssems[bi], add=True)
                    if do_counts:
                        @pl.when(c == 0)
                        def _():
                            chs.append(pltpu.async_copy(
                                ones_v.at[pl.ds(0, CH)],
                                cnt_acc.at[cur.at[1, j]], csem, add=True))
                    if j + 2 < NJ:
                        sh[bi].wait()
                        gh[bi] = pltpu.async_copy(tab.at[cur.at[0, j + 2]],
                                                  bufs[bi], gsems[bi])
                sh[(NJ - 2) % 2].wait()
                sh[(NJ - 1) % 2].wait()
                if do_counts:
                    @pl.when(c == 0)
                    def _():
                        for ch in chs:
                            ch.wait()
                hpre.wait()
                return carry
            lax.fori_loop(0, NG, gbody, 0)

            plsc.subcore_barrier()

            bcol = b * DB
            for j in range(RPT // DB):
                pltpu.sync_copy(acc.at[pl.ds(row0 + j * DB, DB)],
                                out.at[pl.ds(row0 + j * DB, DB),
                                       pl.ds(bcol, DB)])
            if with_counts and bb == 0:
                @pl.when(c == 0)
                def _():
                    pltpu.sync_copy(cnt_acc.at[pl.ds(row0, RPT)],
                                    cnt_out.at[pl.ds(row0, RPT)])
            if bb + 1 < nbpc:
                plsc.subcore_barrier()

    return segsum


def _aggregate(table, idxb, nb, with_counts):
    res = _make_segsum(nb, with_counts)(table, idxb)
    if with_counts:
        agg, cnt = res
        return agg, cnt[:, None]
    return res[0] if isinstance(res, (tuple, list)) else res


def _matmul_body(a_ref, wt_ref, o_ref):
    o_ref[...] = jnp.dot(a_ref[...], wt_ref[...],
                         preferred_element_type=jnp.float32)


def _dense_matmul(a, wT):
    din, h = wT.shape
    return pl.pallas_call(
        _matmul_body,
        grid=(N_NODES // BN,),
        in_specs=[
            pl.BlockSpec((BN, din), lambda i: (i, 0)),
            pl.BlockSpec((din, h), lambda i: (0, 0)),
        ],
        out_specs=pl.BlockSpec((BN, h), lambda i: (i, 0)),
        out_shape=jax.ShapeDtypeStruct((N_NODES, h), jnp.float32),
    )(a, wT)


def _combine1_body(cnt_ref, agg_ref, xr_ref, wlt_ref, bl_ref, o_ref):
    inv = 1.0 / jnp.clip(cnt_ref[...], 1.0, None)
    mean = agg_ref[...] * inv
    h = jnp.dot(mean, wlt_ref[...], preferred_element_type=jnp.float32)
    o_ref[...] = jnp.maximum(h + xr_ref[...] + bl_ref[...], 0.0)


def _combine1(cnt, agg, xr, WlT, bl):
    din, h = WlT.shape
    return pl.pallas_call(
        _combine1_body,
        grid=(N_NODES // BN,),
        in_specs=[
            pl.BlockSpec((BN, 1), lambda i: (i, 0)),
            pl.BlockSpec((BN, din), lambda i: (i, 0)),
            pl.BlockSpec((BN, h), lambda i: (i, 0)),
            pl.BlockSpec((din, h), lambda i: (0, 0)),
            pl.BlockSpec((1, h), lambda i: (0, 0)),
        ],
        out_specs=pl.BlockSpec((BN, h), lambda i: (i, 0)),
        out_shape=jax.ShapeDtypeStruct((N_NODES, h), jnp.float32),
    )(cnt, agg, xr, WlT, bl)


def _combine2_body(cnt_ref, agg_ref, xr_ref, wlt_ref, bl_ref,
                   wlint_ref, blin_ref, o_ref):
    inv = 1.0 / jnp.clip(cnt_ref[...], 1.0, None)
    mean = agg_ref[...] * inv
    h2 = jnp.dot(mean, wlt_ref[...], preferred_element_type=jnp.float32)
    h2 = jnp.maximum(h2 + xr_ref[...] + bl_ref[...], 0.0)
    out = jnp.dot(h2, wlint_ref[...], preferred_element_type=jnp.float32)
    o_ref[...] = out + blin_ref[...]


def _combine2(cnt, agg, xr, Wl2T, bl2, WlinT, blin):
    din, h = Wl2T.shape
    out = WlinT.shape[1]
    return pl.pallas_call(
        _combine2_body,
        grid=(N_NODES // BN,),
        in_specs=[
            pl.BlockSpec((BN, 1), lambda i: (i, 0)),
            pl.BlockSpec((BN, din), lambda i: (i, 0)),
            pl.BlockSpec((BN, h), lambda i: (i, 0)),
            pl.BlockSpec((din, h), lambda i: (0, 0)),
            pl.BlockSpec((1, h), lambda i: (0, 0)),
            pl.BlockSpec((h, out), lambda i: (0, 0)),
            pl.BlockSpec((1, out), lambda i: (0, 0)),
        ],
        out_specs=pl.BlockSpec((BN, out), lambda i: (i, 0)),
        out_shape=jax.ShapeDtypeStruct((N_NODES, out), jnp.float32),
    )(cnt, agg, xr, Wl2T, bl2, WlinT, blin)


def kernel(x, edge_index, Wl1, bl1, Wr1, Wl2, bl2, Wr2, Wlin, blin):
    src = edge_index[0].astype(jnp.int32)
    dst = edge_index[1].astype(jnp.int32)

    def idx_blocks(nb):
        offs = jnp.arange(nb, dtype=jnp.int32)[:, None]
        srcb = src[None, :] * nb + offs
        dstb = jnp.broadcast_to(dst[None, :], (nb, N_EDGES_TOTAL))
        comb = jnp.stack([srcb, dstb], axis=1)
        comb = comb.reshape(nb, 2, NS, NG, NJ, CH)
        return comb.transpose(0, 2, 3, 1, 4, 5)

    nb1 = x.shape[1] // DB
    xt = x.reshape(nb1 * N_NODES, DB)
    xr1 = _dense_matmul(x, Wr1.T)
    agg1, cnt = _aggregate(xt, idx_blocks(nb1), nb1, True)
    h1 = _combine1(cnt, agg1, xr1, Wl1.T, bl1[None, :])

    nb2 = h1.shape[1] // DB
    ht = h1.reshape(nb2 * N_NODES, DB)
    xr2 = _dense_matmul(h1, Wr2.T)
    agg2 = _aggregate(ht, idx_blocks(nb2), nb2, False)
    return _combine2(cnt, agg2, xr2, Wl2.T, bl2[None, :], Wlin.T, blin[None, :])


# speedup vs baseline: 7.4235x; 1.0393x over previous
import functools

import jax
import jax.numpy as jnp
from jax import lax
from jax.experimental import pallas as pl
from jax.experimental.pallas import tpu as pltpu
from jax.experimental.pallas import tpu_sc as plsc

N_NODES = 10000
N_EDGES_TOTAL = 160000
NPAD = 10240
DB = 128
NS = 16
NC = 2
RPT = NPAD // NS
EPT = N_EDGES_TOTAL // NS
CH = 125
NJ = 10
NG = EPT // (NJ * CH)
BN = 1000


@functools.lru_cache(maxsize=None)
def _make_segsum(nb, with_counts):
    nbpc = nb // NC
    mesh = plsc.VectorSubcoreMesh(core_axis_name="c", subcore_axis_name="s")

    out_type = [jax.ShapeDtypeStruct((NPAD, nb * DB), jnp.float32)]
    if with_counts:
        out_type.append(jax.ShapeDtypeStruct((NPAD,), jnp.float32))

    scratch_types = [
        pltpu.VMEM((2, 2, NJ, CH), jnp.int32),
        pltpu.VMEM((CH, DB), jnp.float32),
        pltpu.VMEM((CH, DB), jnp.float32),
        pltpu.VMEM((128,), jnp.float32),
        pltpu.VMEM_SHARED((NPAD, DB), jnp.float32),
        pltpu.VMEM_SHARED((NPAD,), jnp.float32),
        pltpu.SemaphoreType.DMA,
        pltpu.SemaphoreType.DMA,
        pltpu.SemaphoreType.DMA,
        pltpu.SemaphoreType.DMA,
        pltpu.SemaphoreType.DMA,
        pltpu.SemaphoreType.DMA,
    ]

    @functools.partial(pl.kernel, mesh=mesh, out_type=tuple(out_type),
                       scratch_types=scratch_types)
    def segsum(tab, idxh, *refs):
        if with_counts:
            out, cnt_out = refs[0], refs[1]
            scratch = refs[2:]
        else:
            out = refs[0]
            scratch = refs[1:]
        (idxbuf, rows0, rows1, ones_v, acc, cnt_acc,
         gsem0, gsem1, ssem0, ssem1, csem, isem) = scratch
        bufs = (rows0, rows1)
        gsems = (gsem0, gsem1)
        ssems = (ssem0, ssem1)

        c = lax.axis_index("c")
        s = lax.axis_index("s")
        row0 = s * RPT

        for kk in range(8):
            ones_v[pl.ds(kk * 16, 16)] = jnp.ones((16,), jnp.float32)

        def zero_acc():
            def zrow(r, carry):
                for kk in range(DB // 16):
                    rows0[r, pl.ds(kk * 16, 16)] = jnp.zeros((16,), jnp.float32)
                return carry
            lax.fori_loop(0, CH, zrow, 0)
            for j in range(RPT // 40):
                pltpu.sync_copy(rows0.at[pl.ds(0, 40)],
                                acc.at[pl.ds(row0 + j * 40, 40)])

        def zero_cnt():
            for j in range(RPT // DB):
                pltpu.sync_copy(rows0.at[0], cnt_acc.at[pl.ds(row0 + j * DB, DB)])

        zero_acc()
        if with_counts:
            @pl.when(c == 0)
            def _():
                zero_cnt()

        for bb in range(nbpc):
            if bb > 0:
                zero_acc()
            plsc.subcore_barrier()

            b = c * nbpc + bb
            do_counts = with_counts and bb == 0

            pltpu.sync_copy(idxh.at[b, s, 0], idxbuf.at[0])

            def gbody(g, carry):
                p = lax.rem(g, 2)
                gg = jnp.minimum(g + 1, NG - 1)
                hpre = pltpu.async_copy(idxh.at[b, s, gg],
                                        idxbuf.at[1 - p], isem)
                cur = idxbuf.at[p]
                gh = [None, None]
                sh = [None, None]
                chs = []
                for jj in range(2):
                    gh[jj] = pltpu.async_copy(tab.at[cur.at[0, jj]],
                                              bufs[jj], gsems[jj])
                for j in range(NJ):
                    bi = j % 2
                    gh[bi].wait()
                    sh[bi] = pltpu.async_copy(bufs[bi],
                                              acc.at[cur.at[1, j]],
                                              ssems[bi], add=True)
                    if do_counts:
                        @pl.when(c == 0)
                        def _():
                            chs.append(pltpu.async_copy(
                                ones_v.at[pl.ds(0, CH)],
                                cnt_acc.at[cur.at[1, j]], csem, add=True))
                    if j + 2 < NJ:
                        sh[bi].wait()
                        gh[bi] = pltpu.async_copy(tab.at[cur.at[0, j + 2]],
                                                  bufs[bi], gsems[bi])
                sh[(NJ - 2) % 2].wait()
                sh[(NJ - 1) % 2].wait()
                if do_counts:
                    @pl.when(c == 0)
                    def _():
                        for ch in chs:
                            ch.wait()
                hpre.wait()
                return carry
            lax.fori_loop(0, NG, gbody, 0)

            plsc.subcore_barrier()

            bcol = b * DB
            for j in range(RPT // DB):
                pltpu.sync_copy(acc.at[pl.ds(row0 + j * DB, DB)],
                                out.at[pl.ds(row0 + j * DB, DB),
                                       pl.ds(bcol, DB)])
            if with_counts and bb == 0:
                @pl.when(c == 0)
                def _():
                    pltpu.sync_copy(cnt_acc.at[pl.ds(row0, RPT)],
                                    cnt_out.at[pl.ds(row0, RPT)])
            if bb + 1 < nbpc:
                plsc.subcore_barrier()

    return segsum


def _aggregate(table, idxb, nb, with_counts):
    res = _make_segsum(nb, with_counts)(table, idxb)
    if with_counts:
        agg, cnt = res
        return agg, cnt[:, None]
    return res[0] if isinstance(res, (tuple, list)) else res


def _matmul_body(a_ref, wt_ref, o_ref):
    o_ref[...] = jnp.dot(a_ref[...], wt_ref[...],
                         preferred_element_type=jnp.float32)


def _dense_matmul(a, wT):
    din, h = wT.shape
    return pl.pallas_call(
        _matmul_body,
        grid=(N_NODES // BN,),
        in_specs=[
            pl.BlockSpec((BN, din), lambda i: (i, 0)),
            pl.BlockSpec((din, h), lambda i: (0, 0)),
        ],
        out_specs=pl.BlockSpec((BN, h), lambda i: (i, 0)),
        out_shape=jax.ShapeDtypeStruct((N_NODES, h), jnp.float32),
    )(a, wT)


def _combine1_body(cnt_ref, agg_ref, xr_ref, wlt_ref, bl_ref, o_ref):
    inv = 1.0 / jnp.clip(cnt_ref[...], 1.0, None)
    mean = agg_ref[...] * inv
    h = jnp.dot(mean, wlt_ref[...], preferred_element_type=jnp.float32)
    o_ref[...] = jnp.maximum(h + xr_ref[...] + bl_ref[...], 0.0)


def _combine1(cnt, agg, xr, WlT, bl):
    din, h = WlT.shape
    return pl.pallas_call(
        _combine1_body,
        grid=(N_NODES // BN,),
        in_specs=[
            pl.BlockSpec((BN, 1), lambda i: (i, 0)),
            pl.BlockSpec((BN, din), lambda i: (i, 0)),
            pl.BlockSpec((BN, h), lambda i: (i, 0)),
            pl.BlockSpec((din, h), lambda i: (0, 0)),
            pl.BlockSpec((1, h), lambda i: (0, 0)),
        ],
        out_specs=pl.BlockSpec((BN, h), lambda i: (i, 0)),
        out_shape=jax.ShapeDtypeStruct((N_NODES, h), jnp.float32),
    )(cnt, agg, xr, WlT, bl)


def _combine2_body(cnt_ref, agg_ref, xr_ref, wlt_ref, bl_ref,
                   wlint_ref, blin_ref, o_ref):
    inv = 1.0 / jnp.clip(cnt_ref[...], 1.0, None)
    mean = agg_ref[...] * inv
    h2 = jnp.dot(mean, wlt_ref[...], preferred_element_type=jnp.float32)
    h2 = jnp.maximum(h2 + xr_ref[...] + bl_ref[...], 0.0)
    out = jnp.dot(h2, wlint_ref[...], preferred_element_type=jnp.float32)
    o_ref[...] = out + blin_ref[...]


def _combine2(cnt, agg, xr, Wl2T, bl2, WlinT, blin):
    din, h = Wl2T.shape
    out = WlinT.shape[1]
    return pl.pallas_call(
        _combine2_body,
        grid=(N_NODES // BN,),
        in_specs=[
            pl.BlockSpec((BN, 1), lambda i: (i, 0)),
            pl.BlockSpec((BN, din), lambda i: (i, 0)),
            pl.BlockSpec((BN, h), lambda i: (i, 0)),
            pl.BlockSpec((din, h), lambda i: (0, 0)),
            pl.BlockSpec((1, h), lambda i: (0, 0)),
            pl.BlockSpec((h, out), lambda i: (0, 0)),
            pl.BlockSpec((1, out), lambda i: (0, 0)),
        ],
        out_specs=pl.BlockSpec((BN, out), lambda i: (i, 0)),
        out_shape=jax.ShapeDtypeStruct((N_NODES, out), jnp.float32),
    )(cnt, agg, xr, Wl2T, bl2, WlinT, blin)


def kernel(x, edge_index, Wl1, bl1, Wr1, Wl2, bl2, Wr2, Wlin, blin):
    src = edge_index[0].astype(jnp.int32)
    dst = edge_index[1].astype(jnp.int32)

    def idx_blocks(nb):
        offs = jnp.arange(nb, dtype=jnp.int32)[:, None]
        srcb = src[None, :] * nb + offs
        dstb = jnp.broadcast_to(dst[None, :], (nb, N_EDGES_TOTAL))
        comb = jnp.stack([srcb, dstb], axis=1)
        comb = comb.reshape(nb, 2, NS, NG, NJ, CH)
        return comb.transpose(0, 2, 3, 1, 4, 5)

    nb1 = x.shape[1] // DB
    xt = x.reshape(nb1 * N_NODES, DB)
    xr1 = _dense_matmul(x, Wr1.T)
    agg1, cnt = _aggregate(xt, idx_blocks(nb1), nb1, True)
    h1 = _combine1(cnt, agg1, xr1, Wl1.T, bl1[None, :])

    nb2 = h1.shape[1] // DB
    ht = h1.reshape(nb2 * N_NODES, DB)
    xr2 = _dense_matmul(h1, Wr2.T)
    agg2 = _aggregate(ht, idx_blocks(nb2), nb2, False)
    return _combine2(cnt, agg2, xr2, Wl2.T, bl2[None, :], Wlin.T, blin[None, :])
